# baseline (speedup 1.0000x reference)
import functools

import jax
import jax.numpy as jnp
from jax import lax
from jax.experimental import pallas as pl
from jax.experimental.pallas import tpu as pltpu
from jax.experimental.pallas import tpu_sc as plsc

N = 10000
E = 320000
F_IN = 128
HID = 128
CLS = 32
K = 20
ALPHA = 0.1

HALF = 16
NC, NS, L = 2, 16, 16
RPT = 632
NROW = NS * RPT
CHUNK = 128
NCH_PT = 158
PAIRS = NCH_PT // 2
EPAD = NS * NCH_PT * CHUNK

_mesh = plsc.VectorSubcoreMesh(
    core_axis_name="c", subcore_axis_name="s", num_cores=NC, num_subcores=NS
)


def _rsqrt16(d):
    i = lax.bitcast_convert_type(d, jnp.int32)
    y = lax.bitcast_convert_type(jnp.int32(0x5F3759DF) - (i >> 1), jnp.float32)
    for _ in range(3):
        y = y * (1.5 - 0.5 * d * y * y)
    return y


def _row_loop(body, n=RPT, unroll=4):
    def outer(k, _):
        for u in range(unroll):
            body(k * unroll + u)
        return 0

    assert n % unroll == 0
    lax.fori_loop(0, n // unroll, outer, 0)


def _sc_propagate(h0_hbm, src_hbm, dst_hbm, out_hbm, hs_hbm,
                  sagg, idx_s, idx_d, hbuf, dinv, h0a, tmp,
                  rows_a, rows_b, sem_ga, sem_gb, sem_sa, sem_sb):
    c = lax.axis_index("c")
    s = lax.axis_index("s")
    r0 = s * RPT

    pltpu.sync_copy(h0_hbm.at[c, pl.ds(r0, RPT)], hbuf)
    pltpu.sync_copy(src_hbm.at[s], idx_s)
    pltpu.sync_copy(dst_hbm.at[s], idx_d)

    bias = (c * NROW).astype(jnp.int32)

    def _bias_idx(j):
        for q in range(CHUNK // L):
            idx_s[j, pl.ds(q * L, L)] = idx_s[j, pl.ds(q * L, L)] + bias

    _row_loop(_bias_idx, NCH_PT, unroll=2)

    def _init_h0a(i):
        h0a[i, :] = hbuf[i, :] * ALPHA

    _row_loop(_init_h0a)

    def _zero_tmp(i):
        tmp[i, :] = jnp.zeros((L,), jnp.float32)

    _row_loop(_zero_tmp)

    def _ones_rows(i):
        rows_a[i, :] = jnp.ones((L,), jnp.float32)

    _row_loop(_ones_rows, CHUNK)

    pltpu.sync_copy(tmp, sagg.at[pl.ds(r0, RPT)])
    plsc.subcore_barrier()

    def _deg_chunk(j, _):
        pltpu.sync_copy(rows_a, sagg.at[idx_d.at[j]], add=True)
        return 0

    lax.fori_loop(0, NCH_PT, _deg_chunk, 0)
    plsc.subcore_barrier()

    pltpu.sync_copy(sagg.at[pl.ds(r0, RPT)], tmp)

    def _dinv(i):
        dinv[i, :] = _rsqrt16(tmp[i, :] + 1.0)

    _row_loop(_dinv)

    def _hs0(i):
        tmp[i, :] = dinv[i, :] * hbuf[i, :]

    _row_loop(_hs0)

    def _gather(j, buf, sem):
        return pltpu.async_copy(hs_hbm.at[idx_s.at[j]], buf, sem)

    def _scatter(j, buf, sem):
        return pltpu.async_copy(buf, sagg.at[idx_d.at[j]], sem, add=True)

    def _iter(_, carry):
        pltpu.sync_copy(tmp, hs_hbm.at[pl.ds(c * NROW + r0, RPT)])
        pltpu.sync_copy(tmp, sagg.at[pl.ds(r0, RPT)])
        plsc.subcore_barrier()

        _gather(0, rows_a, sem_ga)

        def _pair(p, _):
            j = 2 * p
            pltpu.make_async_copy(hs_hbm.at[idx_s.at[j]], rows_a, sem_ga).wait()
            gb = _gather(j + 1, rows_b, sem_gb)
            sa = _scatter(j, rows_a, sem_sa)
            gb.wait()
            sa.wait()
            _gather(jnp.minimum(j + 2, NCH_PT - 1), rows_a, sem_ga)
            sb = _scatter(j + 1, rows_b, sem_sb)
            sb.wait()
            return 0

        lax.fori_loop(0, PAIRS, _pair, 0)
        pltpu.make_async_copy(
            hs_hbm.at[idx_s.at[NCH_PT - 1]], rows_a, sem_ga).wait()
        plsc.subcore_barrier()

        pltpu.sync_copy(sagg.at[pl.ds(r0, RPT)], tmp)

        def _update(i):
            h = (1.0 - ALPHA) * dinv[i, :] * tmp[i, :] + h0a[i, :]
            hbuf[i, :] = h
            tmp[i, :] = dinv[i, :] * h

        _row_loop(_update)
        return carry

    lax.fori_loop(0, K, _iter, 0)

    pltpu.sync_copy(hbuf, out_hbm.at[c, pl.ds(r0, RPT)])


_sc_prop_call = functools.partial(
    pl.kernel,
    out_type=(jax.ShapeDtypeStruct((NC, NROW, HALF), jnp.float32),
              jax.ShapeDtypeStruct((NC * NROW, HALF), jnp.float32)),
    mesh=_mesh,
    compiler_params=pltpu.CompilerParams(use_tc_tiling_on_sc=False),
    scratch_types=[
        pltpu.VMEM_SHARED((NROW, HALF), jnp.float32),
        pltpu.VMEM((NCH_PT, CHUNK), jnp.int32),
        pltpu.VMEM((NCH_PT, CHUNK), jnp.int32),
        pltpu.VMEM((RPT, HALF), jnp.float32),
        pltpu.VMEM((RPT, HALF), jnp.float32),
        pltpu.VMEM((RPT, HALF), jnp.float32),
        pltpu.VMEM((RPT, HALF), jnp.float32),
        pltpu.VMEM((CHUNK, HALF), jnp.float32),
        pltpu.VMEM((CHUNK, HALF), jnp.float32),
        pltpu.SemaphoreType.DMA,
        pltpu.SemaphoreType.DMA,
        pltpu.SemaphoreType.DMA,
        pltpu.SemaphoreType.DMA,
    ],
)(_sc_propagate)


def _mlp_body(x_ref, w1_ref, b1_ref, w2_ref, b2_ref, out_ref):
    h = jnp.dot(x_ref[...], w1_ref[...], preferred_element_type=jnp.float32)
    h = jnp.maximum(h + b1_ref[...], 0.0)
    out_ref[...] = (
        jnp.dot(h, w2_ref[...], preferred_element_type=jnp.float32)
        + b2_ref[...]
    )


def _softmax_body(h_ref, out_ref):
    h = h_ref[...]
    m = jnp.max(h, axis=1, keepdims=True)
    e = jnp.exp(h - m)
    out_ref[...] = h - m - jnp.log(jnp.sum(e, axis=1, keepdims=True))


_BLK = 1000
_GRID = N // _BLK


def kernel(x, edge_index, W1, b1, W2, b2):
    pad = jnp.full((EPAD - E,), N, jnp.int32)
    src = jnp.concatenate([edge_index[0].astype(jnp.int32), pad])
    dst = jnp.concatenate([edge_index[1].astype(jnp.int32), pad])
    src = src.reshape(NS, NCH_PT, CHUNK)
    dst = dst.reshape(NS, NCH_PT, CHUNK)

    h0 = pl.pallas_call(
        _mlp_body,
        grid=(_GRID,),
        in_specs=[
            pl.BlockSpec((_BLK, F_IN), lambda i: (i, 0)),
            pl.BlockSpec((F_IN, HID), lambda i: (0, 0)),
            pl.BlockSpec((1, HID), lambda i: (0, 0)),
            pl.BlockSpec((HID, CLS), lambda i: (0, 0)),
            pl.BlockSpec((1, CLS), lambda i: (0, 0)),
        ],
        out_specs=pl.BlockSpec((_BLK, CLS), lambda i: (i, 0)),
        out_shape=jax.ShapeDtypeStruct((N, CLS), jnp.float32),
    )(x, W1, b1.reshape(1, HID), W2, b2.reshape(1, CLS))

    h0_split = jnp.stack([h0[:, :HALF], h0[:, HALF:]])
    h0_split = jnp.pad(h0_split, ((0, 0), (0, NROW - N), (0, 0)))
    hf, _ = _sc_prop_call(h0_split, src, dst)
    h_final = jnp.concatenate([hf[0, :N], hf[1, :N]], axis=1)

    return pl.pallas_call(
        _softmax_body,
        grid=(_GRID,),
        in_specs=[pl.BlockSpec((_BLK, CLS), lambda i: (i, 0))],
        out_specs=pl.BlockSpec((_BLK, CLS), lambda i: (i, 0)),
        out_shape=jax.ShapeDtypeStruct((N, CLS), jnp.float32),
    )(h_final)


# speedup vs baseline: 2.8163x; 2.8163x over previous
import functools

import jax
import jax.numpy as jnp
from jax import lax
from jax.experimental import pallas as pl
from jax.experimental.pallas import tpu as pltpu
from jax.experimental.pallas import tpu_sc as plsc

N = 10000
E = 320000
F_IN = 128
HID = 128
CLS = 32
K = 20
ALPHA = 0.1

HALF = 16
NC, NS, L = 2, 16, 16
RPT = 632
NROW = NS * RPT
CHUNK = 128
NCH_PT = 160
GROUPS = NCH_PT // 4
EPAD = NS * NCH_PT * CHUNK

_mesh = plsc.VectorSubcoreMesh(
    core_axis_name="c", subcore_axis_name="s", num_cores=NC, num_subcores=NS
)


def _rsqrt16(d):
    i = lax.bitcast_convert_type(d, jnp.int32)
    y = lax.bitcast_convert_type(jnp.int32(0x5F3759DF) - (i >> 1), jnp.float32)
    for _ in range(3):
        y = y * (1.5 - 0.5 * d * y * y)
    return y


def _row_loop(body, n=RPT, unroll=4):
    def outer(k, _):
        for u in range(unroll):
            body(k * unroll + u)
        return 0

    assert n % unroll == 0
    lax.fori_loop(0, n // unroll, outer, 0)


def _sc_propagate(h0_hbm, src_hbm, dst_hbm, out_hbm,
                  shs, sagg, idx_s, idx_d, hbuf, dinv, h0a, tmp,
                  rows_a, rows_b, rows_c, rows_d,
                  sem_ga, sem_gb, sem_gc, sem_gd,
                  sem_sa, sem_sb, sem_sc, sem_sd):
    c = lax.axis_index("c")
    s = lax.axis_index("s")
    r0 = s * RPT

    pltpu.sync_copy(h0_hbm.at[c, pl.ds(r0, RPT)], hbuf)
    pltpu.sync_copy(src_hbm.at[s], idx_s)
    pltpu.sync_copy(dst_hbm.at[s], idx_d)

    def _init_h0a(i):
        h0a[i, :] = hbuf[i, :] * ALPHA

    _row_loop(_init_h0a)

    def _zero_tmp(i):
        tmp[i, :] = jnp.zeros((L,), jnp.float32)

    _row_loop(_zero_tmp)

    def _ones_rows(i):
        rows_a[i, :] = jnp.ones((L,), jnp.float32)

    _row_loop(_ones_rows, CHUNK)

    pltpu.sync_copy(tmp, sagg.at[pl.ds(r0, RPT)])
    pltpu.sync_copy(tmp, shs.at[pl.ds(r0, RPT)])
    plsc.subcore_barrier()

    def _deg_chunk(j, _):
        pltpu.sync_copy(rows_a, sagg.at[idx_d.at[j]], add=True)
        return 0

    lax.fori_loop(0, NCH_PT, _deg_chunk, 0)
    plsc.subcore_barrier()

    pltpu.sync_copy(sagg.at[pl.ds(r0, RPT)], tmp)

    def _dinv(i):
        dinv[i, :] = _rsqrt16(tmp[i, :] + 1.0)

    _row_loop(_dinv)

    def _hs0(i):
        tmp[i, :] = dinv[i, :] * hbuf[i, :]

    _row_loop(_hs0)

    def _gather(j, buf, sem):
        return pltpu.async_copy(shs.at[idx_s.at[j]], buf, sem)

    def _scatter(j, buf, sem):
        return pltpu.async_copy(buf, sagg.at[idx_d.at[j]], sem, add=True)

    def _iter(_, carry):
        pltpu.sync_copy(tmp, shs.at[pl.ds(r0, RPT)])
        pltpu.sync_copy(tmp, sagg.at[pl.ds(r0, RPT)])
        plsc.subcore_barrier()

        ring = ((rows_a, sem_ga, sem_sa), (rows_b, sem_gb, sem_sb),
                (rows_c, sem_gc, sem_sc), (rows_d, sem_gd, sem_sd))
        for u, (buf, gsem, _ssem) in enumerate(ring):
            _gather(u, buf, gsem)

        def _group(g, _):
            j0 = 4 * g
            for u, (buf, gsem, ssem) in enumerate(ring):
                pltpu.make_async_copy(
                    shs.at[idx_s.at[j0 + u]], buf, gsem).wait()
                _scatter(j0 + u, buf, ssem)
            for u, (buf, gsem, ssem) in enumerate(ring):
                pltpu.make_async_copy(
                    buf, sagg.at[idx_d.at[j0 + u]], ssem).wait()
                _gather(jnp.minimum(j0 + 4 + u, NCH_PT - 1), buf, gsem)
            return 0

        lax.fori_loop(0, GROUPS, _group, 0)
        for u, (buf, gsem, _ssem) in enumerate(ring):
            pltpu.make_async_copy(
                shs.at[idx_s.at[NCH_PT - 1]], buf, gsem).wait()
        plsc.subcore_barrier()

        pltpu.sync_copy(sagg.at[pl.ds(r0, RPT)], tmp)

        def _update(i):
            h = (1.0 - ALPHA) * dinv[i, :] * tmp[i, :] + h0a[i, :]
            hbuf[i, :] = h
            tmp[i, :] = dinv[i, :] * h

        _row_loop(_update)
        return carry

    lax.fori_loop(0, K, _iter, 0)

    pltpu.sync_copy(hbuf, out_hbm.at[c, pl.ds(r0, RPT)])


_sc_prop_call = functools.partial(
    pl.kernel,
    out_type=jax.ShapeDtypeStruct((NC, NROW, HALF), jnp.float32),
    mesh=_mesh,
    compiler_params=pltpu.CompilerParams(use_tc_tiling_on_sc=False),
    scratch_types=[
        pltpu.VMEM_SHARED((NROW, HALF), jnp.float32),
        pltpu.VMEM_SHARED((NROW, HALF), jnp.float32),
        pltpu.VMEM((NCH_PT, CHUNK), jnp.int32),
        pltpu.VMEM((NCH_PT, CHUNK), jnp.int32),
        pltpu.VMEM((RPT, HALF), jnp.float32),
        pltpu.VMEM((RPT, HALF), jnp.float32),
        pltpu.VMEM((RPT, HALF), jnp.float32),
        pltpu.VMEM((RPT, HALF), jnp.float32),
        pltpu.VMEM((CHUNK, HALF), jnp.float32),
        pltpu.VMEM((CHUNK, HALF), jnp.float32),
        pltpu.VMEM((CHUNK, HALF), jnp.float32),
        pltpu.VMEM((CHUNK, HALF), jnp.float32),
    ] + [pltpu.SemaphoreType.DMA] * 8,
)(_sc_propagate)


def _mlp_body(x_ref, w1_ref, b1_ref, w2_ref, b2_ref, out_ref):
    h = jnp.dot(x_ref[...], w1_ref[...], preferred_element_type=jnp.float32)
    h = jnp.maximum(h + b1_ref[...], 0.0)
    out_ref[...] = (
        jnp.dot(h, w2_ref[...], preferred_element_type=jnp.float32)
        + b2_ref[...]
    )


def _softmax_body(h_ref, out_ref):
    h = h_ref[...]
    m = jnp.max(h, axis=1, keepdims=True)
    e = jnp.exp(h - m)
    out_ref[...] = h - m - jnp.log(jnp.sum(e, axis=1, keepdims=True))


_BLK = 1000
_GRID = N // _BLK


def kernel(x, edge_index, W1, b1, W2, b2):
    pad = jnp.full((EPAD - E,), N, jnp.int32)
    src = jnp.concatenate([edge_index[0].astype(jnp.int32), pad])
    dst = jnp.concatenate([edge_index[1].astype(jnp.int32), pad])
    src = src.reshape(NS, NCH_PT, CHUNK)
    dst = dst.reshape(NS, NCH_PT, CHUNK)

    h0 = pl.pallas_call(
        _mlp_body,
        grid=(_GRID,),
        in_specs=[
            pl.BlockSpec((_BLK, F_IN), lambda i: (i, 0)),
            pl.BlockSpec((F_IN, HID), lambda i: (0, 0)),
            pl.BlockSpec((1, HID), lambda i: (0, 0)),
            pl.BlockSpec((HID, CLS), lambda i: (0, 0)),
            pl.BlockSpec((1, CLS), lambda i: (0, 0)),
        ],
        out_specs=pl.BlockSpec((_BLK, CLS), lambda i: (i, 0)),
        out_shape=jax.ShapeDtypeStruct((N, CLS), jnp.float32),
    )(x, W1, b1.reshape(1, HID), W2, b2.reshape(1, CLS))

    h0_split = jnp.stack([h0[:, :HALF], h0[:, HALF:]])
    h0_split = jnp.pad(h0_split, ((0, 0), (0, NROW - N), (0, 0)))
    hf = _sc_prop_call(h0_split, src, dst)
    h_final = jnp.concatenate([hf[0, :N], hf[1, :N]], axis=1)

    return pl.pallas_call(
        _softmax_body,
        grid=(_GRID,),
        in_specs=[pl.BlockSpec((_BLK, CLS), lambda i: (i, 0))],
        out_specs=pl.BlockSpec((_BLK, CLS), lambda i: (i, 0)),
        out_shape=jax.ShapeDtypeStruct((N, CLS), jnp.float32),
    )(h_final)


# speedup vs baseline: 3.2717x; 1.1617x over previous
import functools

import jax
import jax.numpy as jnp
from jax import lax
from jax.experimental import pallas as pl
from jax.experimental.pallas import tpu as pltpu
from jax.experimental.pallas import tpu_sc as plsc

N = 10000
E = 320000
F_IN = 128
HID = 128
CLS = 32
K = 20
ALPHA = 0.1

HALF = 16
NC, NS, L = 2, 16, 16
RPT = 632
NROW = NS * RPT
CHUNK = 128
NCH_PT = 158
PAIRS = NCH_PT // 2
EPAD = NS * NCH_PT * CHUNK

_mesh = plsc.VectorSubcoreMesh(
    core_axis_name="c", subcore_axis_name="s", num_cores=NC, num_subcores=NS
)


def _rsqrt16(d):
    i = lax.bitcast_convert_type(d, jnp.int32)
    y = lax.bitcast_convert_type(jnp.int32(0x5F3759DF) - (i >> 1), jnp.float32)
    for _ in range(3):
        y = y * (1.5 - 0.5 * d * y * y)
    return y


def _row_loop(body, n=RPT, unroll=4):
    def outer(k, _):
        for u in range(unroll):
            body(k * unroll + u)
        return 0

    assert n % unroll == 0
    lax.fori_loop(0, n // unroll, outer, 0)


def _sc_propagate(h0_hbm, src_hbm, dst_hbm, out_hbm,
                  shs, sagg, idx_s, idx_d, hbuf, dinv, h0a, tmp,
                  rows_a, rows_b, sem_ga, sem_gb, sem_sa, sem_sb):
    c = lax.axis_index("c")
    s = lax.axis_index("s")
    r0 = s * RPT

    pltpu.sync_copy(h0_hbm.at[c, pl.ds(r0, RPT)], hbuf)
    pltpu.sync_copy(src_hbm.at[s], idx_s)
    pltpu.sync_copy(dst_hbm.at[s], idx_d)

    def _init_h0a(i):
        h0a[i, :] = hbuf[i, :] * ALPHA

    _row_loop(_init_h0a)

    def _zero_tmp(i):
        tmp[i, :] = jnp.zeros((L,), jnp.float32)

    _row_loop(_zero_tmp)

    def _ones_rows(i):
        rows_a[i, :] = jnp.ones((L,), jnp.float32)

    _row_loop(_ones_rows, CHUNK)

    pltpu.sync_copy(tmp, sagg.at[pl.ds(r0, RPT)])
    pltpu.sync_copy(tmp, shs.at[pl.ds(r0, RPT)])
    plsc.subcore_barrier()

    def _deg_chunk(j, _):
        pltpu.sync_copy(rows_a, sagg.at[idx_d.at[j]], add=True)
        return 0

    lax.fori_loop(0, NCH_PT, _deg_chunk, 0)
    plsc.subcore_barrier()

    pltpu.sync_copy(sagg.at[pl.ds(r0, RPT)], tmp)

    def _dinv(i):
        dinv[i, :] = _rsqrt16(tmp[i, :] + 1.0)

    _row_loop(_dinv)

    def _hs0(i):
        tmp[i, :] = dinv[i, :] * hbuf[i, :]

    _row_loop(_hs0)

    def _gather(j, buf, sem):
        return pltpu.async_copy(shs.at[idx_s.at[j]], buf, sem)

    def _scatter(j, buf, sem):
        return pltpu.async_copy(buf, sagg.at[idx_d.at[j]], sem, add=True)

    def _iter(_, carry):
        pltpu.sync_copy(tmp, shs.at[pl.ds(r0, RPT)])
        pltpu.sync_copy(tmp, sagg.at[pl.ds(r0, RPT)])
        plsc.subcore_barrier()

        def _wait_g(j, buf, sem):
            pltpu.make_async_copy(shs.at[idx_s.at[j]], buf, sem).wait()

        def _wait_s(j, buf, sem):
            pltpu.make_async_copy(buf, sagg.at[idx_d.at[j]], sem).wait()

        _gather(0, rows_a, sem_ga)
        _wait_g(0, rows_a, sem_ga)
        _scatter(0, rows_a, sem_sa)
        _gather(1, rows_b, sem_gb)

        def _pair(p, _):
            j = 2 * p + 1
            _wait_g(j, rows_b, sem_gb)
            _scatter(j, rows_b, sem_sb)
            _wait_s(j - 1, rows_a, sem_sa)
            _gather(j + 1, rows_a, sem_ga)
            _wait_g(j + 1, rows_a, sem_ga)
            _scatter(j + 1, rows_a, sem_sa)
            _wait_s(j, rows_b, sem_sb)
            _gather(jnp.minimum(j + 2, NCH_PT - 1), rows_b, sem_gb)
            return 0

        lax.fori_loop(0, PAIRS - 1, _pair, 0)
        _wait_g(NCH_PT - 1, rows_b, sem_gb)
        _scatter(NCH_PT - 1, rows_b, sem_sb)
        _wait_s(NCH_PT - 2, rows_a, sem_sa)
        _wait_s(NCH_PT - 1, rows_b, sem_sb)
        plsc.subcore_barrier()

        pltpu.sync_copy(sagg.at[pl.ds(r0, RPT)], tmp)

        def _update(i):
            h = (1.0 - ALPHA) * dinv[i, :] * tmp[i, :] + h0a[i, :]
            hbuf[i, :] = h
            tmp[i, :] = dinv[i, :] * h

        _row_loop(_update)
        return carry

    lax.fori_loop(0, K, _iter, 0)

    pltpu.sync_copy(hbuf, out_hbm.at[c, pl.ds(r0, RPT)])


_sc_prop_call = functools.partial(
    pl.kernel,
    out_type=jax.ShapeDtypeStruct((NC, NROW, HALF), jnp.float32),
    mesh=_mesh,
    compiler_params=pltpu.CompilerParams(use_tc_tiling_on_sc=False),
    scratch_types=[
        pltpu.VMEM_SHARED((NROW, HALF), jnp.float32),
        pltpu.VMEM_SHARED((NROW, HALF), jnp.float32),
        pltpu.VMEM((NCH_PT, CHUNK), jnp.int32),
        pltpu.VMEM((NCH_PT, CHUNK), jnp.int32),
        pltpu.VMEM((RPT, HALF), jnp.float32),
        pltpu.VMEM((RPT, HALF), jnp.float32),
        pltpu.VMEM((RPT, HALF), jnp.float32),
        pltpu.VMEM((RPT, HALF), jnp.float32),
        pltpu.VMEM((CHUNK, HALF), jnp.float32),
        pltpu.VMEM((CHUNK, HALF), jnp.float32),
    ] + [pltpu.SemaphoreType.DMA] * 4,
)(_sc_propagate)


def _mlp_body(x_ref, w1_ref, b1_ref, w2_ref, b2_ref, out_ref):
    h = jnp.dot(x_ref[...], w1_ref[...], preferred_element_type=jnp.float32)
    h = jnp.maximum(h + b1_ref[...], 0.0)
    out_ref[...] = (
        jnp.dot(h, w2_ref[...], preferred_element_type=jnp.float32)
        + b2_ref[...]
    )


def _softmax_body(h_ref, out_ref):
    h = h_ref[...]
    m = jnp.max(h, axis=1, keepdims=True)
    e = jnp.exp(h - m)
    out_ref[...] = h - m - jnp.log(jnp.sum(e, axis=1, keepdims=True))


_BLK = 1000
_GRID = N // _BLK


def kernel(x, edge_index, W1, b1, W2, b2):
    pad = jnp.full((EPAD - E,), N, jnp.int32)
    src = jnp.concatenate([edge_index[0].astype(jnp.int32), pad])
    dst = jnp.concatenate([edge_index[1].astype(jnp.int32), pad])
    src = src.reshape(NS, NCH_PT, CHUNK)
    dst = dst.reshape(NS, NCH_PT, CHUNK)

    h0 = pl.pallas_call(
        _mlp_body,
        grid=(_GRID,),
        in_specs=[
            pl.BlockSpec((_BLK, F_IN), lambda i: (i, 0)),
            pl.BlockSpec((F_IN, HID), lambda i: (0, 0)),
            pl.BlockSpec((1, HID), lambda i: (0, 0)),
            pl.BlockSpec((HID, CLS), lambda i: (0, 0)),
            pl.BlockSpec((1, CLS), lambda i: (0, 0)),
        ],
        out_specs=pl.BlockSpec((_BLK, CLS), lambda i: (i, 0)),
        out_shape=jax.ShapeDtypeStruct((N, CLS), jnp.float32),
    )(x, W1, b1.reshape(1, HID), W2, b2.reshape(1, CLS))

    h0_split = jnp.stack([h0[:, :HALF], h0[:, HALF:]])
    h0_split = jnp.pad(h0_split, ((0, 0), (0, NROW - N), (0, 0)))
    hf = _sc_prop_call(h0_split, src, dst)
    h_final = jnp.concatenate([hf[0, :N], hf[1, :N]], axis=1)

    return pl.pallas_call(
        _softmax_body,
        grid=(_GRID,),
        in_specs=[pl.BlockSpec((_BLK, CLS), lambda i: (i, 0))],
        out_specs=pl.BlockSpec((_BLK, CLS), lambda i: (i, 0)),
        out_shape=jax.ShapeDtypeStruct((N, CLS), jnp.float32),
    )(h_final)


# speedup vs baseline: 3.2923x; 1.0063x over previous
import functools

import jax
import jax.numpy as jnp
from jax import lax
from jax.experimental import pallas as pl
from jax.experimental.pallas import tpu as pltpu
from jax.experimental.pallas import tpu_sc as plsc

N = 10000
E = 320000
F_IN = 128
HID = 128
CLS = 32
K = 20
ALPHA = 0.1

HALF = 16
NC, NS, L = 2, 16, 16
RPT = 632
NROW = NS * RPT
CHUNK = 128
NCH_PT = 158
PAIRS = NCH_PT // 2
EPAD = NS * NCH_PT * CHUNK

_mesh = plsc.VectorSubcoreMesh(
    core_axis_name="c", subcore_axis_name="s", num_cores=NC, num_subcores=NS
)


def _rsqrt16(d):
    i = lax.bitcast_convert_type(d, jnp.int32)
    y = lax.bitcast_convert_type(jnp.int32(0x5F3759DF) - (i >> 1), jnp.float32)
    for _ in range(3):
        y = y * (1.5 - 0.5 * d * y * y)
    return y


def _row_loop(body, n=RPT, unroll=4):
    def outer(k, _):
        for u in range(unroll):
            body(k * unroll + u)
        return 0

    assert n % unroll == 0
    lax.fori_loop(0, n // unroll, outer, 0)


def _sc_propagate(h0_hbm, src_hbm, dst_hbm, out_hbm,
                  shs, sagg, idx_s, idx_d, hbuf, dinv, h0a, tmp,
                  rows_a, rows_b, sem_ga, sem_gb, sem_sa, sem_sb):
    c = lax.axis_index("c")
    s = lax.axis_index("s")
    r0 = s * RPT

    pltpu.sync_copy(h0_hbm.at[c, pl.ds(r0, RPT)], hbuf)
    pltpu.sync_copy(src_hbm.at[s], idx_s)
    pltpu.sync_copy(dst_hbm.at[s], idx_d)

    def _init_h0a(i):
        h0a[i, :] = hbuf[i, :] * ALPHA

    _row_loop(_init_h0a)

    def _zero_tmp(i):
        tmp[i, :] = jnp.zeros((L,), jnp.float32)

    _row_loop(_zero_tmp)

    def _ones_rows(i):
        rows_a[i, :] = jnp.ones((L,), jnp.float32)

    _row_loop(_ones_rows, CHUNK)

    pltpu.sync_copy(tmp, sagg.at[pl.ds(r0, RPT)])
    pltpu.sync_copy(tmp, shs.at[pl.ds(r0, RPT)])
    plsc.subcore_barrier()

    def _deg_chunk(j, _):
        pltpu.async_copy(rows_a, sagg.at[idx_d.at[j]], sem_sa, add=True)
        return 0

    lax.fori_loop(0, NCH_PT, _deg_chunk, 0)

    def _deg_drain(j, _):
        pltpu.make_async_copy(rows_a, sagg.at[idx_d.at[0]], sem_sa).wait()
        return 0

    lax.fori_loop(0, NCH_PT, _deg_drain, 0)
    plsc.subcore_barrier()

    pltpu.sync_copy(sagg.at[pl.ds(r0, RPT)], tmp)

    def _dinv(i):
        dinv[i, :] = _rsqrt16(tmp[i, :] + 1.0)

    _row_loop(_dinv)

    def _hs0(i):
        tmp[i, :] = dinv[i, :] * hbuf[i, :]

    _row_loop(_hs0)

    def _gather(j, buf, sem):
        return pltpu.async_copy(shs.at[idx_s.at[j]], buf, sem)

    def _scatter(j, buf, sem):
        return pltpu.async_copy(buf, sagg.at[idx_d.at[j]], sem, add=True)

    def _iter(_, carry):
        c1 = pltpu.async_copy(tmp, shs.at[pl.ds(r0, RPT)], sem_ga)
        c2 = pltpu.async_copy(tmp, sagg.at[pl.ds(r0, RPT)], sem_sa)
        c1.wait()
        c2.wait()
        plsc.subcore_barrier()

        def _wait_g(j, buf, sem):
            pltpu.make_async_copy(shs.at[idx_s.at[j]], buf, sem).wait()

        def _wait_s(j, buf, sem):
            pltpu.make_async_copy(buf, sagg.at[idx_d.at[j]], sem).wait()

        _gather(0, rows_a, sem_ga)
        _wait_g(0, rows_a, sem_ga)
        _scatter(0, rows_a, sem_sa)
        _gather(1, rows_b, sem_gb)

        def _pair(p, _):
            j = 2 * p + 1
            _wait_g(j, rows_b, sem_gb)
            _scatter(j, rows_b, sem_sb)
            _wait_s(j - 1, rows_a, sem_sa)
            _gather(j + 1, rows_a, sem_ga)
            _wait_g(j + 1, rows_a, sem_ga)
            _scatter(j + 1, rows_a, sem_sa)
            _wait_s(j, rows_b, sem_sb)
            _gather(jnp.minimum(j + 2, NCH_PT - 1), rows_b, sem_gb)
            return 0

        lax.fori_loop(0, PAIRS - 1, _pair, 0)
        _wait_g(NCH_PT - 1, rows_b, sem_gb)
        _scatter(NCH_PT - 1, rows_b, sem_sb)
        _wait_s(NCH_PT - 2, rows_a, sem_sa)
        _wait_s(NCH_PT - 1, rows_b, sem_sb)
        plsc.subcore_barrier()

        pltpu.sync_copy(sagg.at[pl.ds(r0, RPT)], tmp)

        def _update(i):
            h = (1.0 - ALPHA) * dinv[i, :] * tmp[i, :] + h0a[i, :]
            hbuf[i, :] = h
            tmp[i, :] = dinv[i, :] * h

        _row_loop(_update)
        return carry

    lax.fori_loop(0, K, _iter, 0)

    pltpu.sync_copy(hbuf, out_hbm.at[c, pl.ds(r0, RPT)])


_sc_prop_call = functools.partial(
    pl.kernel,
    out_type=jax.ShapeDtypeStruct((NC, NROW, HALF), jnp.float32),
    mesh=_mesh,
    compiler_params=pltpu.CompilerParams(use_tc_tiling_on_sc=False),
    scratch_types=[
        pltpu.VMEM_SHARED((NROW, HALF), jnp.float32),
        pltpu.VMEM_SHARED((NROW, HALF), jnp.float32),
        pltpu.VMEM((NCH_PT, CHUNK), jnp.int32),
        pltpu.VMEM((NCH_PT, CHUNK), jnp.int32),
        pltpu.VMEM((RPT, HALF), jnp.float32),
        pltpu.VMEM((RPT, HALF), jnp.float32),
        pltpu.VMEM((RPT, HALF), jnp.float32),
        pltpu.VMEM((RPT, HALF), jnp.float32),
        pltpu.VMEM((CHUNK, HALF), jnp.float32),
        pltpu.VMEM((CHUNK, HALF), jnp.float32),
    ] + [pltpu.SemaphoreType.DMA] * 4,
)(_sc_propagate)


def _mlp_body(x_ref, w1_ref, b1_ref, w2_ref, b2_ref, out_ref):
    h = jnp.dot(x_ref[...], w1_ref[...], preferred_element_type=jnp.float32)
    h = jnp.maximum(h + b1_ref[...], 0.0)
    out_ref[...] = (
        jnp.dot(h, w2_ref[...], preferred_element_type=jnp.float32)
        + b2_ref[...]
    )


def _softmax_body(h_ref, out_ref):
    h = h_ref[...]
    m = jnp.max(h, axis=1, keepdims=True)
    e = jnp.exp(h - m)
    out_ref[...] = h - m - jnp.log(jnp.sum(e, axis=1, keepdims=True))


_BLK = 1000
_GRID = N // _BLK


def kernel(x, edge_index, W1, b1, W2, b2):
    pad = jnp.full((EPAD - E,), N, jnp.int32)
    src = jnp.concatenate([edge_index[0].astype(jnp.int32), pad])
    dst = jnp.concatenate([edge_index[1].astype(jnp.int32), pad])
    src = src.reshape(NS, NCH_PT, CHUNK)
    dst = dst.reshape(NS, NCH_PT, CHUNK)

    h0 = pl.pallas_call(
        _mlp_body,
        grid=(_GRID,),
        in_specs=[
            pl.BlockSpec((_BLK, F_IN), lambda i: (i, 0)),
            pl.BlockSpec((F_IN, HID), lambda i: (0, 0)),
            pl.BlockSpec((1, HID), lambda i: (0, 0)),
            pl.BlockSpec((HID, CLS), lambda i: (0, 0)),
            pl.BlockSpec((1, CLS), lambda i: (0, 0)),
        ],
        out_specs=pl.BlockSpec((_BLK, CLS), lambda i: (i, 0)),
        out_shape=jax.ShapeDtypeStruct((N, CLS), jnp.float32),
    )(x, W1, b1.reshape(1, HID), W2, b2.reshape(1, CLS))

    h0_split = jnp.stack([h0[:, :HALF], h0[:, HALF:]])
    h0_split = jnp.pad(h0_split, ((0, 0), (0, NROW - N), (0, 0)))
    hf = _sc_prop_call(h0_split, src, dst)
    h_final = jnp.concatenate([hf[0, :N], hf[1, :N]], axis=1)

    return pl.pallas_call(
        _softmax_body,
        grid=(_GRID,),
        in_specs=[pl.BlockSpec((_BLK, CLS), lambda i: (i, 0))],
        out_specs=pl.BlockSpec((_BLK, CLS), lambda i: (i, 0)),
        out_shape=jax.ShapeDtypeStruct((N, CLS), jnp.float32),
    )(h_final)
